# baseline (speedup 1.0000x reference)
import functools
import math

import jax
import jax.numpy as jnp
from jax import lax
from jax.experimental import pallas as pl
from jax.experimental.pallas import tpu as pltpu
from jax.experimental.pallas import tpu_sc as plsc

N = 10000
F = 128
E = 320000
EPS = 1e-5

NC = 2
NS = 16
NW = NC * NS
EPW = E // NW
CH = 40
NCHUNK = 250
EPWP = NCHUNK * CH
NP = 10240
RPT = NP // NS
DUMP = N

_SC_MESH = plsc.VectorSubcoreMesh(core_axis_name="c", subcore_axis_name="s")


EPC = E // NS
DCH = 80


def _deg_body(ef_hbm, z1_hbm, out_hbm, idxv, onesv, degv, deg_sh):
    c = lax.axis_index("c")
    s = lax.axis_index("s")
    base = c * E + s * EPC
    ones = jnp.ones((16,), jnp.float32)
    for k in range(DCH // 16):
        onesv[pl.ds(k * 16, 16)] = ones

    @pl.when(s == 0)
    def _():
        pltpu.sync_copy(z1_hbm, deg_sh)

    plsc.subcore_barrier()

    def body(j, carry):
        pltpu.sync_copy(ef_hbm.at[pl.ds(base + j * DCH, DCH)], idxv)
        pltpu.sync_copy(onesv, deg_sh.at[idxv], add=True)
        return carry

    lax.fori_loop(0, EPC // DCH, body, 0)
    plsc.subcore_barrier()
    pltpu.sync_copy(deg_sh.at[pl.ds(s * RPT, RPT)], degv)
    pltpu.sync_copy(degv, out_hbm.at[pl.ds(c * NP + s * RPT, RPT)])


_deg_call = functools.partial(
    pl.kernel,
    out_type=jax.ShapeDtypeStruct((2 * NP,), jnp.float32),
    mesh=_SC_MESH,
    scratch_types=[
        pltpu.VMEM((DCH,), jnp.int32),
        pltpu.VMEM((DCH,), jnp.float32),
        pltpu.VMEM((RPT,), jnp.float32),
        pltpu.VMEM_SHARED((NP,), jnp.float32),
    ],
)(_deg_body)


NB = 8


def _agg_body(hw_hbm, eidx_hbm, znf_hbm, out_hbm,
              ibufs, rows, acc_sh, isem, gsem):
    c = lax.axis_index("c")
    s = lax.axis_index("s")
    wid = c * NS + s
    jbase = wid * NCHUNK
    pltpu.sync_copy(znf_hbm.at[pl.ds(s * RPT, RPT)],
                    acc_sh.at[pl.ds(s * RPT, RPT)])
    plsc.subcore_barrier()

    def idx_load(j, b):
        pltpu.async_copy(eidx_hbm.at[jbase + j], ibufs[b], isem)

    def wait_idx():
        pltpu.make_async_copy(eidx_hbm.at[0], ibufs[0], isem).wait()

    def gather(j, b):
        pltpu.async_copy(hw_hbm.at[ibufs[b].at[0]], rows[b], gsem)

    def wait_gather(b):
        pltpu.make_async_copy(hw_hbm.at[pl.ds(0, CH)], rows[b], gsem).wait()

    def scatter(b):
        pltpu.sync_copy(rows[b], acc_sh.at[ibufs[b].at[1]], add=True)

    for k in range(NB):
        idx_load(k, k)
    for k in range(NB - 1):
        wait_idx()
        gather(k, k)

    def step(j, u, guard_hi):
        wait_gather(u)
        scatter(u)
        if guard_hi:
            if j + NB < NCHUNK:
                idx_load(j + NB, u)
            if j + NB - 1 < NCHUNK:
                wait_idx()
                gather(j + NB - 1, (u + NB - 1) % NB)
        else:
            idx_load(j + NB, u)
            wait_idx()
            gather(j + NB - 1, (u + NB - 1) % NB)

    def body(t, carry):
        j0 = NB * t
        for u in range(NB):
            step(j0 + u, u, False)
        return carry

    nloop = (NCHUNK - NB) // NB
    lax.fori_loop(0, nloop, body, 0)
    for j in range(nloop * NB, NCHUNK):
        step(j, j % NB, True)
    plsc.subcore_barrier()
    pltpu.sync_copy(acc_sh.at[pl.ds(s * RPT, RPT)],
                    out_hbm.at[c, pl.ds(s * RPT, RPT)])


_agg_call = functools.partial(
    pl.kernel,
    out_type=jax.ShapeDtypeStruct((NC, NP, F), jnp.float32),
    mesh=_SC_MESH,
    scratch_types=[
        [pltpu.VMEM((2, CH), jnp.int32) for _ in range(NB)],
        [pltpu.VMEM((CH, F), jnp.float32) for _ in range(NB)],
        pltpu.VMEM_SHARED((NP, F), jnp.float32),
        pltpu.SemaphoreType.DMA,
        pltpu.SemaphoreType.DMA,
    ],
)(_agg_body)


def _deg_reduce_body(dp_ref, nsrc_ref, ndst_ref):
    nsrc_ref[...] = lax.rsqrt(jnp.maximum(dp_ref[0:1, :], 1.0))
    ndst_ref[...] = lax.rsqrt(jnp.maximum(dp_ref[1:2, :], 1.0))


_deg_reduce_call = pl.pallas_call(
    _deg_reduce_body,
    out_shape=(jax.ShapeDtypeStruct((1, N), jnp.float32),
               jax.ShapeDtypeStruct((1, N), jnp.float32)),
)

BR = 2000


def _mm0_body(x_ref, nsrc_ref, w_ref, out_ref):
    out_ref[...] = jnp.dot(x_ref[...] * nsrc_ref[...], w_ref[...],
                           preferred_element_type=jnp.float32)


_mm0_call = pl.pallas_call(
    _mm0_body,
    grid=(N // BR,),
    in_specs=[
        pl.BlockSpec((BR, F), lambda i: (i, 0)),
        pl.BlockSpec((BR, 1), lambda i: (i, 0)),
        pl.BlockSpec((F, F), lambda i: (0, 0)),
    ],
    out_specs=pl.BlockSpec((BR, F), lambda i: (i, 0)),
    out_shape=jax.ShapeDtypeStruct((N, F), jnp.float32),
)


def _layer_body(aggp_ref, ndst_ref, nsrc_ref, w_ref, b_ref, out_ref):
    agg = aggp_ref[0] + aggp_ref[1]
    h = jnp.maximum(agg * ndst_ref[...] + b_ref[...], 0.0)
    out_ref[...] = jnp.dot(h * nsrc_ref[...], w_ref[...],
                           preferred_element_type=jnp.float32)


_layer_call = pl.pallas_call(
    _layer_body,
    grid=(N // BR,),
    in_specs=[
        pl.BlockSpec((NC, BR, F), lambda i: (0, i, 0)),
        pl.BlockSpec((BR, 1), lambda i: (i, 0)),
        pl.BlockSpec((BR, 1), lambda i: (i, 0)),
        pl.BlockSpec((F, F), lambda i: (0, 0)),
        pl.BlockSpec((1, F), lambda i: (0, 0)),
    ],
    out_specs=pl.BlockSpec((BR, F), lambda i: (i, 0)),
    out_shape=jax.ShapeDtypeStruct((N, F), jnp.float32),
)

_FINAL_IN_SPECS = [
    pl.BlockSpec((NC, N, F), lambda i: (0, 0, 0)),
    pl.BlockSpec((N, 1), lambda i: (0, 0)),
    pl.BlockSpec((1, F), lambda i: (0, 0)),
    pl.BlockSpec((3 * F, F), lambda i: (0, 0)),
    pl.BlockSpec((1, F), lambda i: (0, 0)),
    pl.BlockSpec((1, F), lambda i: (0, 0)),
    pl.BlockSpec((1, F), lambda i: (0, 0)),
    pl.BlockSpec((1, F), lambda i: (0, 0)),
    pl.BlockSpec((1, 1), lambda i: (0, 0)),
]

_BN_SCALE = 1.0 / math.sqrt(1.0 + EPS)


def _final_body(aggp_ref, ndst_ref, b_ref, w1_ref, b1_ref, g_ref, be_ref,
                w2_ref, b2_ref, out_ref):
    agg = aggp_ref[0] + aggp_ref[1]
    h = jnp.maximum(agg * ndst_ref[...] + b_ref[...], 0.0)
    r_sum = jnp.sum(h, axis=0, keepdims=True)
    r_mean = r_sum * (1.0 / N)
    r_max = jnp.max(h, axis=0, keepdims=True)
    readout = jnp.concatenate([r_sum, r_mean, r_max], axis=1)
    z = jnp.dot(readout, w1_ref[...], preferred_element_type=jnp.float32)
    z = (z + b1_ref[...]) * (_BN_SCALE * g_ref[...]) + be_ref[...]
    z = jnp.maximum(z, 0.0)
    out_ref[...] = jnp.sum(z * w2_ref[...], axis=1, keepdims=True) + b2_ref[...]


_final_call = pl.pallas_call(
    _final_body,
    grid=(1,),
    in_specs=_FINAL_IN_SPECS,
    out_specs=pl.BlockSpec((1, 1), lambda i: (0, 0)),
    out_shape=jax.ShapeDtypeStruct((1, 1), jnp.float32),
)


def kernel(x, edge_index, W0, b0, W1, b1, W2, b2, W3, b3, W4, b4,
           mlpW1, mlpb1, gamma, beta, mlpW2, mlpb2):
    edge_flat = edge_index.astype(jnp.int32).reshape(2 * E)
    pad = EPWP - EPW
    src = jnp.pad(edge_flat[:E].reshape(NW, EPW), ((0, 0), (0, pad)))
    dst = jnp.pad(edge_flat[E:].reshape(NW, EPW), ((0, 0), (0, pad)),
                  constant_values=DUMP)
    eidx = jnp.stack([src.reshape(NW, NCHUNK, CH),
                      dst.reshape(NW, NCHUNK, CH)],
                     axis=2).reshape(NW * NCHUNK, 2, CH)
    zeros_n = jnp.zeros((NP,), jnp.float32)
    zeros_nf = jnp.zeros((NP, F), jnp.float32)

    degp = _deg_call(edge_flat, zeros_n).reshape(2, NP)[:, :N]
    nsrc_row, ndst_row = _deg_reduce_call(degp)
    nsrc = nsrc_row.reshape(N, 1)
    ndst = ndst_row.reshape(N, 1)

    hw = _mm0_call(x, nsrc, W0)
    Ws = [W1, W2, W3, W4]
    bs = [b0, b1, b2, b3]
    for i in range(4):
        aggp = _agg_call(hw, eidx, zeros_nf)
        hw = _layer_call(aggp, ndst, nsrc, Ws[i], bs[i].reshape(1, F))
    aggp = _agg_call(hw, eidx, zeros_nf)
    out = _final_call(aggp, ndst, b4.reshape(1, F),
                      mlpW1, mlpb1.reshape(1, F),
                      gamma.reshape(1, F), beta.reshape(1, F),
                      mlpW2.reshape(1, F), mlpb2.reshape(1, 1))
    return out


# speedup vs baseline: 1.2487x; 1.2487x over previous
import functools
import math

import jax
import jax.numpy as jnp
from jax import lax
from jax.experimental import pallas as pl
from jax.experimental.pallas import tpu as pltpu
from jax.experimental.pallas import tpu_sc as plsc

N = 10000
F = 128
E = 320000
EPS = 1e-5

NC = 2
NS = 16
NW = NC * NS
EPW = E // NW
CH = 80
NCHUNK = 125
EPWP = NCHUNK * CH
NP = 10240
RPT = NP // NS
DUMP = N

_SC_MESH = plsc.VectorSubcoreMesh(core_axis_name="c", subcore_axis_name="s")


DNB = 4


def _deg_body(eidx_hbm, z1_hbm, out_hbm, ibufs, onesv, degv,
              degs_sh, degd_sh, isem, ssem):
    c = lax.axis_index("c")
    s = lax.axis_index("s")
    wid = c * NS + s
    jbase = wid * NCHUNK
    ones = jnp.ones((16,), jnp.float32)
    for k in range(CH // 16):
        onesv[pl.ds(k * 16, 16)] = ones

    @pl.when(s == 0)
    def _():
        pltpu.sync_copy(z1_hbm, degs_sh)
        pltpu.sync_copy(z1_hbm, degd_sh)

    plsc.subcore_barrier()

    def idx_load(j, b):
        pltpu.async_copy(eidx_hbm.at[jbase + j], ibufs[b], isem)

    def wait_idx():
        pltpu.make_async_copy(eidx_hbm.at[0], ibufs[0], isem).wait()

    def drain_scatters():
        pltpu.make_async_copy(onesv, degs_sh.at[pl.ds(0, CH)], ssem).wait()
        pltpu.make_async_copy(onesv, degd_sh.at[pl.ds(0, CH)], ssem).wait()

    def step(j, u, guard_hi):
        wait_idx()
        pltpu.make_async_copy(onesv, degs_sh.at[ibufs[u].at[0]],
                              ssem).start(add=True)
        pltpu.make_async_copy(onesv, degd_sh.at[ibufs[u].at[1]],
                              ssem).start(add=True)
        if guard_hi:
            if j >= DNB - 1:
                drain_scatters()
            if DNB <= j + 1 < NCHUNK:
                idx_load(j + 1, (j + 1) % DNB)
        else:
            drain_scatters()
            idx_load(j + 1, (u + 1) % DNB)

    for k in range(DNB):
        idx_load(k, k)
    for j in range(DNB - 1):
        step(j, j, True)

    def body(t, carry):
        j0 = DNB - 1 + (DNB * t)
        for u in range(DNB):
            step(j0 + u, (DNB - 1 + u) % DNB, False)
        return carry

    nloop = (NCHUNK - DNB) // DNB
    lax.fori_loop(0, nloop, body, 0)
    for j in range(DNB - 1 + nloop * DNB, NCHUNK):
        step(j, j % DNB, True)
    for j in range(NCHUNK - DNB + 1, NCHUNK):
        drain_scatters()
    plsc.subcore_barrier()
    pltpu.sync_copy(degs_sh.at[pl.ds(s * RPT, RPT)], degv)
    pltpu.sync_copy(degv, out_hbm.at[pl.ds((2 * c) * NP + s * RPT, RPT)])
    pltpu.sync_copy(degd_sh.at[pl.ds(s * RPT, RPT)], degv)
    pltpu.sync_copy(degv, out_hbm.at[pl.ds((2 * c + 1) * NP + s * RPT, RPT)])


_deg_call = functools.partial(
    pl.kernel,
    out_type=jax.ShapeDtypeStruct((4 * NP,), jnp.float32),
    mesh=_SC_MESH,
    scratch_types=[
        [pltpu.VMEM((2, CH), jnp.int32) for _ in range(DNB)],
        pltpu.VMEM((CH,), jnp.float32),
        pltpu.VMEM((RPT,), jnp.float32),
        pltpu.VMEM_SHARED((NP,), jnp.float32),
        pltpu.VMEM_SHARED((NP,), jnp.float32),
        pltpu.SemaphoreType.DMA,
        pltpu.SemaphoreType.DMA,
    ],
)(_deg_body)


NB = 4


def _agg_body(hw_hbm, eidx_hbm, znf_hbm, out_hbm,
              ibufs, rows, acc_sh, isem, gsem):
    c = lax.axis_index("c")
    s = lax.axis_index("s")
    wid = c * NS + s
    jbase = wid * NCHUNK
    pltpu.sync_copy(znf_hbm.at[pl.ds(s * RPT, RPT)],
                    acc_sh.at[pl.ds(s * RPT, RPT)])
    plsc.subcore_barrier()

    def idx_load(j, b):
        pltpu.async_copy(eidx_hbm.at[jbase + j], ibufs[b], isem)

    def wait_idx():
        pltpu.make_async_copy(eidx_hbm.at[0], ibufs[0], isem).wait()

    def gather(j, b):
        pltpu.async_copy(hw_hbm.at[ibufs[b].at[0]], rows[b], gsem)

    def wait_gather(b):
        pltpu.make_async_copy(hw_hbm.at[pl.ds(0, CH)], rows[b], gsem).wait()

    def scatter(b):
        pltpu.sync_copy(rows[b], acc_sh.at[ibufs[b].at[1]], add=True)

    for k in range(NB):
        idx_load(k, k)
    for k in range(NB - 1):
        wait_idx()
        gather(k, k)

    def step(j, u, guard_hi):
        wait_gather(u)
        scatter(u)
        if guard_hi:
            if j + NB < NCHUNK:
                idx_load(j + NB, u)
            if j + NB - 1 < NCHUNK:
                wait_idx()
                gather(j + NB - 1, (u + NB - 1) % NB)
        else:
            idx_load(j + NB, u)
            wait_idx()
            gather(j + NB - 1, (u + NB - 1) % NB)

    def body(t, carry):
        j0 = NB * t
        for u in range(NB):
            step(j0 + u, u, False)
        return carry

    nloop = (NCHUNK - NB) // NB
    lax.fori_loop(0, nloop, body, 0)
    for j in range(nloop * NB, NCHUNK):
        step(j, j % NB, True)
    plsc.subcore_barrier()
    pltpu.sync_copy(acc_sh.at[pl.ds(s * RPT, RPT)],
                    out_hbm.at[c, pl.ds(s * RPT, RPT)])


_agg_call = functools.partial(
    pl.kernel,
    out_type=jax.ShapeDtypeStruct((NC, NP, F), jnp.float32),
    mesh=_SC_MESH,
    scratch_types=[
        [pltpu.VMEM((2, CH), jnp.int32) for _ in range(NB)],
        [pltpu.VMEM((CH, F), jnp.float32) for _ in range(NB)],
        pltpu.VMEM_SHARED((NP, F), jnp.float32),
        pltpu.SemaphoreType.DMA,
        pltpu.SemaphoreType.DMA,
    ],
)(_agg_body)


def _deg_reduce_body(dp_ref, nsrc_ref, ndst_ref):
    s_cnt = dp_ref[0, 0:1, :] + dp_ref[1, 0:1, :]
    d_cnt = dp_ref[0, 1:2, :] + dp_ref[1, 1:2, :]
    nsrc_ref[...] = lax.rsqrt(jnp.maximum(s_cnt, 1.0))
    ndst_ref[...] = lax.rsqrt(jnp.maximum(d_cnt, 1.0))


_deg_reduce_call = pl.pallas_call(
    _deg_reduce_body,
    out_shape=(jax.ShapeDtypeStruct((1, N), jnp.float32),
               jax.ShapeDtypeStruct((1, N), jnp.float32)),
)

BR = 2000


def _mm0_body(x_ref, nsrc_ref, w_ref, out_ref):
    out_ref[...] = jnp.dot(x_ref[...] * nsrc_ref[...], w_ref[...],
                           preferred_element_type=jnp.float32)


_mm0_call = pl.pallas_call(
    _mm0_body,
    grid=(N // BR,),
    in_specs=[
        pl.BlockSpec((BR, F), lambda i: (i, 0)),
        pl.BlockSpec((BR, 1), lambda i: (i, 0)),
        pl.BlockSpec((F, F), lambda i: (0, 0)),
    ],
    out_specs=pl.BlockSpec((BR, F), lambda i: (i, 0)),
    out_shape=jax.ShapeDtypeStruct((N, F), jnp.float32),
)


def _layer_body(aggp_ref, ndst_ref, nsrc_ref, w_ref, b_ref, out_ref):
    agg = aggp_ref[0] + aggp_ref[1]
    h = jnp.maximum(agg * ndst_ref[...] + b_ref[...], 0.0)
    out_ref[...] = jnp.dot(h * nsrc_ref[...], w_ref[...],
                           preferred_element_type=jnp.float32)


_layer_call = pl.pallas_call(
    _layer_body,
    grid=(N // BR,),
    in_specs=[
        pl.BlockSpec((NC, BR, F), lambda i: (0, i, 0)),
        pl.BlockSpec((BR, 1), lambda i: (i, 0)),
        pl.BlockSpec((BR, 1), lambda i: (i, 0)),
        pl.BlockSpec((F, F), lambda i: (0, 0)),
        pl.BlockSpec((1, F), lambda i: (0, 0)),
    ],
    out_specs=pl.BlockSpec((BR, F), lambda i: (i, 0)),
    out_shape=jax.ShapeDtypeStruct((N, F), jnp.float32),
)

_FINAL_IN_SPECS = [
    pl.BlockSpec((NC, N, F), lambda i: (0, 0, 0)),
    pl.BlockSpec((N, 1), lambda i: (0, 0)),
    pl.BlockSpec((1, F), lambda i: (0, 0)),
    pl.BlockSpec((3 * F, F), lambda i: (0, 0)),
    pl.BlockSpec((1, F), lambda i: (0, 0)),
    pl.BlockSpec((1, F), lambda i: (0, 0)),
    pl.BlockSpec((1, F), lambda i: (0, 0)),
    pl.BlockSpec((1, F), lambda i: (0, 0)),
    pl.BlockSpec((1, 1), lambda i: (0, 0)),
]

_BN_SCALE = 1.0 / math.sqrt(1.0 + EPS)


def _final_body(aggp_ref, ndst_ref, b_ref, w1_ref, b1_ref, g_ref, be_ref,
                w2_ref, b2_ref, out_ref):
    agg = aggp_ref[0] + aggp_ref[1]
    h = jnp.maximum(agg * ndst_ref[...] + b_ref[...], 0.0)
    r_sum = jnp.sum(h, axis=0, keepdims=True)
    r_mean = r_sum * (1.0 / N)
    r_max = jnp.max(h, axis=0, keepdims=True)
    readout = jnp.concatenate([r_sum, r_mean, r_max], axis=1)
    z = jnp.dot(readout, w1_ref[...], preferred_element_type=jnp.float32)
    z = (z + b1_ref[...]) * (_BN_SCALE * g_ref[...]) + be_ref[...]
    z = jnp.maximum(z, 0.0)
    out_ref[...] = jnp.sum(z * w2_ref[...], axis=1, keepdims=True) + b2_ref[...]


_final_call = pl.pallas_call(
    _final_body,
    grid=(1,),
    in_specs=_FINAL_IN_SPECS,
    out_specs=pl.BlockSpec((1, 1), lambda i: (0, 0)),
    out_shape=jax.ShapeDtypeStruct((1, 1), jnp.float32),
)


def kernel(x, edge_index, W0, b0, W1, b1, W2, b2, W3, b3, W4, b4,
           mlpW1, mlpb1, gamma, beta, mlpW2, mlpb2):
    edge_flat = edge_index.astype(jnp.int32).reshape(2 * E)
    pad = EPWP - EPW
    src = jnp.pad(edge_flat[:E].reshape(NW, EPW), ((0, 0), (0, pad)))
    dst = jnp.pad(edge_flat[E:].reshape(NW, EPW), ((0, 0), (0, pad)),
                  constant_values=DUMP)
    eidx = jnp.stack([src.reshape(NW, NCHUNK, CH),
                      dst.reshape(NW, NCHUNK, CH)],
                     axis=2).reshape(NW * NCHUNK, 2, CH)
    zeros_n = jnp.zeros((NP,), jnp.float32)
    zeros_nf = jnp.zeros((NP, F), jnp.float32)

    degp = _deg_call(eidx, zeros_n).reshape(2, 2, NP)[:, :, :N]
    nsrc_row, ndst_row = _deg_reduce_call(degp)
    nsrc = nsrc_row.reshape(N, 1)
    ndst = ndst_row.reshape(N, 1)

    hw = _mm0_call(x, nsrc, W0)
    Ws = [W1, W2, W3, W4]
    bs = [b0, b1, b2, b3]
    for i in range(4):
        aggp = _agg_call(hw, eidx, zeros_nf)
        hw = _layer_call(aggp, ndst, nsrc, Ws[i], bs[i].reshape(1, F))
    aggp = _agg_call(hw, eidx, zeros_nf)
    out = _final_call(aggp, ndst, b4.reshape(1, F),
                      mlpW1, mlpb1.reshape(1, F),
                      gamma.reshape(1, F), beta.reshape(1, F),
                      mlpW2.reshape(1, F), mlpb2.reshape(1, 1))
    return out
